# baseline (speedup 1.0000x reference)
import functools

import jax
import jax.numpy as jnp
import numpy as np
from jax import lax
from jax.experimental import pallas as pl
from jax.experimental.pallas import tpu as pltpu
from jax.experimental.pallas import tpu_sc as plsc

N, E, DIN, H, C = 10000, 320000, 128, 128, 64
NC, NS, L = 2, 16, 16
NW = NC * NS
KBY = {H: 64, C: 128}
PHBY = {H: 2, C: 1}
RPS = 624
TAIL = N - RPS * NS


def _interleave_perm(n):
    p = np.empty((n,), np.int32)
    for g in range(0, n, 32):
        p[g + 0:g + 32:2] = g + np.arange(16)
        p[g + 1:g + 32:2] = g + 16 + np.arange(16)
    return p


_PERM_H = _interleave_perm(H)
_PERM_C = _interleave_perm(C)


def _mm_body(x_ref, w_ref, o_ref):
    o_ref[...] = jnp.dot(x_ref[...], w_ref[...],
                         preferred_element_type=jnp.float32
                         ).astype(jnp.bfloat16)


def _matmul_bf16(x, w):
    m, k = x.shape
    n = w.shape[1]
    bm = 1000
    return pl.pallas_call(
        _mm_body,
        grid=(m // bm,),
        in_specs=[pl.BlockSpec((bm, k), lambda i: (i, 0)),
                  pl.BlockSpec((k, n), lambda i: (0, 0))],
        out_specs=pl.BlockSpec((bm, n), lambda i: (i, 0)),
        out_shape=jax.ShapeDtypeStruct((m, n), jnp.bfloat16),
    )(x, w)


def _l2_body(p_ref, b_ref, w_ref, o_ref):
    t = p_ref[0] + p_ref[1] + b_ref[...]
    h = jnp.where(t > 0, t, jnp.exp(jnp.minimum(t, 0.0)) - 1.0)
    o_ref[...] = jnp.dot(h, w_ref[...], preferred_element_type=jnp.float32
                         ).astype(jnp.bfloat16)


def _layer2(p, b1, w2p):
    bm = 1000
    return pl.pallas_call(
        _l2_body,
        grid=(N // bm,),
        in_specs=[pl.BlockSpec((2, bm, H), lambda i: (0, i, 0)),
                  pl.BlockSpec((1, H), lambda i: (0, 0)),
                  pl.BlockSpec((H, C), lambda i: (0, 0))],
        out_specs=pl.BlockSpec((bm, C), lambda i: (i, 0)),
        out_shape=jax.ShapeDtypeStruct((N, C), jnp.bfloat16),
    )(p, b1.reshape(1, H), w2p)


def _sm_body(p_ref, b_ref, o_ref):
    s = p_ref[0] + p_ref[1] + b_ref[...]
    m = jnp.max(s, axis=-1, keepdims=True)
    e = jnp.exp(s - m)
    o_ref[...] = e / jnp.sum(e, axis=-1, keepdims=True)


def _softmax(p, b2):
    bm = 1000
    return pl.pallas_call(
        _sm_body,
        grid=(N // bm,),
        in_specs=[pl.BlockSpec((2, bm, C), lambda i: (0, i, 0)),
                  pl.BlockSpec((1, C), lambda i: (0, 0))],
        out_specs=pl.BlockSpec((bm, C), lambda i: (i, 0)),
        out_shape=jax.ShapeDtypeStruct((N, C), jnp.float32),
    )(p, b2.reshape(1, C))


def _make_sc_scatter(d):
    K = KBY[d]
    NCHUNK = E // K
    CPW = NCHUNK // NW
    XTRA = NCHUNK - CPW * NW
    PH = PHBY[d]
    CPP = CPW // PH
    assert CPP % 2 == 0 and CPP * PH == CPW and XTRA <= NW
    mesh = plsc.VectorSubcoreMesh(core_axis_name="c", subcore_axis_name="s")

    @functools.partial(
        pl.kernel,
        out_type=jax.ShapeDtypeStruct((NC, N, d), jnp.float32),
        mesh=mesh,
        scratch_types=[
            pltpu.VMEM_SHARED((N, d), jnp.float32),
            pltpu.VMEM((CPP + 1, K), jnp.int32),
            pltpu.VMEM((CPP + 1, K), jnp.int32),
            pltpu.VMEM((CPP + 1, K), jnp.float32),
            pltpu.VMEM((2, K, d // 2), jnp.int32),
            pltpu.VMEM((2, K, d), jnp.float32),
            pltpu.SemaphoreType.DMA,
            pltpu.SemaphoreType.DMA,
            pltpu.SemaphoreType.DMA,
            pltpu.SemaphoreType.DMA,
        ],
        compiler_params=pltpu.CompilerParams(use_tc_tiling_on_sc=False),
    )
    def sc_scatter(h_hbm, src_hbm, dst_hbm, ew_hbm, z_hbm, out_hbm,
                   acc, src_v, dst_v, ew_v, rbf_v, rf_v, g0, g1, s0, s1):
        cid = lax.axis_index("c")
        sid = lax.axis_index("s")
        wid = sid * NC + cid

        r0 = sid * RPS
        pltpu.sync_copy(z_hbm.at[pl.ds(r0, RPS)], acc.at[pl.ds(r0, RPS)])

        @pl.when(sid == 0)
        def _():
            pltpu.sync_copy(z_hbm.at[pl.ds(RPS * NS, TAIL)],
                            acc.at[pl.ds(RPS * NS, TAIL)])

        plsc.subcore_barrier()

        def scale(b, c):
            @plsc.parallel_loop(0, K // L, step=1)
            def _(g):
                e0 = g * L
                w16 = ew_v[c, pl.ds(e0, L)]
                for l in range(L):
                    w = jnp.full((L,), w16[l])
                    for j in range(d // 32):
                        v = rbf_v[b, e0 + l, pl.ds(j * L, L)]
                        lo = lax.bitcast_convert_type(v << 16, jnp.float32)
                        hi = lax.bitcast_convert_type(v & jnp.int32(-65536),
                                                      jnp.float32)
                        rf_v[b, e0 + l, pl.ds(j * 32, L)] = lo * w
                        rf_v[b, e0 + l, pl.ds(j * 32 + L, L)] = hi * w

        def gather_start(b, c, sem):
            pltpu.async_copy(h_hbm.at[src_v.at[c]], rbf_v.at[b], sem)

        def gather_wait(b, c, sem):
            pltpu.make_async_copy(h_hbm.at[src_v.at[c]], rbf_v.at[b],
                                  sem).wait()

        def scatter_start(b, c, sem):
            pltpu.async_copy(rf_v.at[b], acc.at[dst_v.at[c]], sem, add=True)

        def scatter_wait(b, c, sem):
            pltpu.make_async_copy(rf_v.at[b], acc.at[dst_v.at[c]],
                                  sem).wait()

        for p in range(PH):
            c0 = wid * CPW + p * CPP
            pltpu.sync_copy(src_hbm.at[pl.ds(c0, CPP)],
                            src_v.at[pl.ds(0, CPP)])
            pltpu.sync_copy(dst_hbm.at[pl.ds(c0, CPP)],
                            dst_v.at[pl.ds(0, CPP)])
            pltpu.sync_copy(ew_hbm.at[pl.ds(c0, CPP)],
                            ew_v.at[pl.ds(0, CPP)])
            has_tail = (wid < XTRA) if p == PH - 1 else (wid < 0)
            nch = jnp.where(has_tail, CPP + 1, CPP)

            @pl.when(has_tail)
            def _():
                xc = NW * CPW + wid
                pltpu.sync_copy(src_hbm.at[pl.ds(xc, 1)],
                                src_v.at[pl.ds(CPP, 1)])
                pltpu.sync_copy(dst_hbm.at[pl.ds(xc, 1)],
                                dst_v.at[pl.ds(CPP, 1)])
                pltpu.sync_copy(ew_hbm.at[pl.ds(xc, 1)],
                                ew_v.at[pl.ds(CPP, 1)])

            gather_start(0, 0, g0)
            gather_start(1, 1, g1)

            def pair_body(i, _):
                c = 2 * i
                gather_wait(0, c, g0)
                scale(0, c)
                scatter_start(0, c, s0)
                gather_wait(1, c + 1, g1)
                scale(1, c + 1)
                scatter_start(1, c + 1, s1)

                @pl.when(c + 2 < nch)
                def _():
                    scatter_wait(0, c, s0)
                    gather_start(0, c + 2, g0)

                @pl.when(c + 3 < nch)
                def _():
                    scatter_wait(1, c + 1, s1)
                    gather_start(1, c + 3, g1)

                return 0

            lax.fori_loop(0, CPP // 2, pair_body, 0)

            @pl.when(has_tail)
            def _():
                c = CPP
                gather_wait(0, c, g0)
                scale(0, c)
                scatter_start(0, c, s0)
                scatter_wait(0, c, s0)

            @pl.when(jnp.logical_not(has_tail))
            def _():
                scatter_wait(0, CPP - 2, s0)

            scatter_wait(1, CPP - 1, s1)

        plsc.subcore_barrier()
        pltpu.sync_copy(acc.at[pl.ds(r0, RPS)], out_hbm.at[cid, pl.ds(r0, RPS)])

        @pl.when(sid == 0)
        def _():
            pltpu.sync_copy(acc.at[pl.ds(RPS * NS, TAIL)],
                            out_hbm.at[cid, pl.ds(RPS * NS, TAIL)])

    return sc_scatter


_sc_scatter_h = _make_sc_scatter(H)
_sc_scatter_c = _make_sc_scatter(C)


def kernel(x, edge_index, edge_weight, W1, b1, W2, b2):
    src = edge_index[0].astype(jnp.int32)
    dst = edge_index[1].astype(jnp.int32)
    ew = edge_weight.astype(jnp.float32)
    kh, kc = KBY[H], KBY[C]
    zh = jnp.zeros((N, H), jnp.float32)
    zc = jnp.zeros((N, C), jnp.float32)
    w1p = W1[:, _PERM_H]
    w2p = W2[:, _PERM_C]

    h1 = _matmul_bf16(x, w1p)
    h1i = lax.bitcast_convert_type(h1.reshape(N, H // 2, 2), jnp.int32)
    p1 = _sc_scatter_h(h1i, src.reshape(-1, kh), dst.reshape(-1, kh),
                       ew.reshape(-1, kh), zh)
    h2 = _layer2(p1, b1, w2p)
    h2i = lax.bitcast_convert_type(h2.reshape(N, C // 2, 2), jnp.int32)
    p2 = _sc_scatter_c(h2i, src.reshape(-1, kc), dst.reshape(-1, kc),
                       ew.reshape(-1, kc), zc)
    return _softmax(p2, b2)


# speedup vs baseline: 1.5679x; 1.5679x over previous
import functools

import jax
import jax.numpy as jnp
from jax import lax
from jax.experimental import pallas as pl
from jax.experimental.pallas import tpu as pltpu
from jax.experimental.pallas import tpu_sc as plsc

N, E, DIN, H, C = 10000, 320000, 128, 128, 64
NC, NS, L = 2, 16, 16
NW = NC * NS
KBY = {H: 128, C: 128}
PHBY = {H: 3, C: 1}
RPS = 624
TAIL = N - RPS * NS


def _mm_body(x_ref, w_ref, o_ref):
    o_ref[...] = jnp.dot(x_ref[...], w_ref[...],
                         preferred_element_type=jnp.float32)


def _matmul(x, w):
    m, k = x.shape
    n = w.shape[1]
    bm = 1000
    return pl.pallas_call(
        _mm_body,
        grid=(m // bm,),
        in_specs=[pl.BlockSpec((bm, k), lambda i: (i, 0)),
                  pl.BlockSpec((k, n), lambda i: (0, 0))],
        out_specs=pl.BlockSpec((bm, n), lambda i: (i, 0)),
        out_shape=jax.ShapeDtypeStruct((m, n), jnp.float32),
    )(x, w)


def _l2_body(p_ref, b_ref, w_ref, o_ref):
    t = p_ref[0] + p_ref[1] + b_ref[...]
    h = jnp.where(t > 0, t, jnp.exp(jnp.minimum(t, 0.0)) - 1.0)
    o_ref[...] = jnp.dot(h, w_ref[...], preferred_element_type=jnp.float32)


def _layer2(p, b1, w2):
    bm = 1000
    return pl.pallas_call(
        _l2_body,
        grid=(N // bm,),
        in_specs=[pl.BlockSpec((2, bm, H), lambda i: (0, i, 0)),
                  pl.BlockSpec((1, H), lambda i: (0, 0)),
                  pl.BlockSpec((H, C), lambda i: (0, 0))],
        out_specs=pl.BlockSpec((bm, C), lambda i: (i, 0)),
        out_shape=jax.ShapeDtypeStruct((N, C), jnp.float32),
    )(p, b1.reshape(1, H), w2)


def _sm_body(p_ref, b_ref, o_ref):
    s = p_ref[0] + p_ref[1] + b_ref[...]
    m = jnp.max(s, axis=-1, keepdims=True)
    e = jnp.exp(s - m)
    o_ref[...] = e / jnp.sum(e, axis=-1, keepdims=True)


def _softmax(p, b2):
    bm = 1000
    return pl.pallas_call(
        _sm_body,
        grid=(N // bm,),
        in_specs=[pl.BlockSpec((2, bm, C), lambda i: (0, i, 0)),
                  pl.BlockSpec((1, C), lambda i: (0, 0))],
        out_specs=pl.BlockSpec((bm, C), lambda i: (i, 0)),
        out_shape=jax.ShapeDtypeStruct((N, C), jnp.float32),
    )(p, b2.reshape(1, C))


def _make_sc_scatter(d):
    K = KBY[d]
    NCHUNK = E // K
    CPW = NCHUNK // NW
    XTRA = NCHUNK - CPW * NW
    PH = PHBY[d]
    CPP = CPW // PH
    assert CPP % 2 == 0 and CPP * PH == CPW and XTRA <= NW
    mesh = plsc.VectorSubcoreMesh(core_axis_name="c", subcore_axis_name="s")

    @functools.partial(
        pl.kernel,
        out_type=jax.ShapeDtypeStruct((NC, N, d), jnp.float32),
        mesh=mesh,
        scratch_types=[
            pltpu.VMEM_SHARED((N, d), jnp.float32),
            pltpu.VMEM((CPP + 1, K), jnp.int32),
            pltpu.VMEM((CPP + 1, K), jnp.int32),
            pltpu.VMEM((CPP + 1, K), jnp.float32),
            pltpu.VMEM((2, K, d), jnp.float32),
            pltpu.SemaphoreType.DMA,
            pltpu.SemaphoreType.DMA,
            pltpu.SemaphoreType.DMA,
            pltpu.SemaphoreType.DMA,
        ],
        compiler_params=pltpu.CompilerParams(use_tc_tiling_on_sc=False),
    )
    def sc_scatter(h_hbm, src_hbm, dst_hbm, ew_hbm, z_hbm, out_hbm,
                   acc, src_v, dst_v, ew_v, rows_v, g0, g1, s0, s1):
        cid = lax.axis_index("c")
        sid = lax.axis_index("s")
        wid = sid * NC + cid

        r0 = sid * RPS
        pltpu.sync_copy(z_hbm.at[pl.ds(r0, RPS)], acc.at[pl.ds(r0, RPS)])

        @pl.when(sid == 0)
        def _():
            pltpu.sync_copy(z_hbm.at[pl.ds(RPS * NS, TAIL)],
                            acc.at[pl.ds(RPS * NS, TAIL)])

        plsc.subcore_barrier()

        def scale(b, c):
            @plsc.parallel_loop(0, K // L, step=1)
            def _(g):
                e0 = g * L
                w16 = ew_v[c, pl.ds(e0, L)]
                for l in range(L):
                    w = jnp.full((L,), w16[l])
                    for j in range(d // L):
                        rows_v[b, e0 + l, pl.ds(j * L, L)] = (
                            rows_v[b, e0 + l, pl.ds(j * L, L)] * w)

        def gather_start(b, c, sem):
            pltpu.async_copy(h_hbm.at[src_v.at[c]], rows_v.at[b], sem)

        def gather_wait(b, c, sem):
            pltpu.make_async_copy(h_hbm.at[src_v.at[c]], rows_v.at[b],
                                  sem).wait()

        def scatter_start(b, c, sem):
            pltpu.async_copy(rows_v.at[b], acc.at[dst_v.at[c]], sem, add=True)

        def scatter_wait(b, c, sem):
            pltpu.make_async_copy(rows_v.at[b], acc.at[dst_v.at[c]],
                                  sem).wait()

        for p in range(PH):
            c0 = wid * CPW + p * CPP
            pltpu.sync_copy(src_hbm.at[pl.ds(c0, CPP)],
                            src_v.at[pl.ds(0, CPP)])
            pltpu.sync_copy(dst_hbm.at[pl.ds(c0, CPP)],
                            dst_v.at[pl.ds(0, CPP)])
            pltpu.sync_copy(ew_hbm.at[pl.ds(c0, CPP)],
                            ew_v.at[pl.ds(0, CPP)])
            has_tail = (wid < XTRA) if p == PH - 1 else (wid < 0)
            nch = jnp.where(has_tail, CPP + 1, CPP)

            @pl.when(has_tail)
            def _():
                xc = NW * CPW + wid
                pltpu.sync_copy(src_hbm.at[pl.ds(xc, 1)],
                                src_v.at[pl.ds(CPP, 1)])
                pltpu.sync_copy(dst_hbm.at[pl.ds(xc, 1)],
                                dst_v.at[pl.ds(CPP, 1)])
                pltpu.sync_copy(ew_hbm.at[pl.ds(xc, 1)],
                                ew_v.at[pl.ds(CPP, 1)])

            gather_start(0, 0, g0)
            gather_start(1, 1, g1)

            def pair_body(i, _):
                c = 2 * i
                gather_wait(0, c, g0)
                scale(0, c)
                scatter_start(0, c, s0)
                gather_wait(1, c + 1, g1)
                scale(1, c + 1)
                scatter_start(1, c + 1, s1)

                @pl.when(c + 2 < nch)
                def _():
                    scatter_wait(0, c, s0)
                    gather_start(0, c + 2, g0)

                @pl.when(c + 3 < nch)
                def _():
                    scatter_wait(1, c + 1, s1)
                    gather_start(1, c + 3, g1)

                return 0

            lax.fori_loop(0, CPP // 2, pair_body, 0)

            @pl.when(has_tail)
            def _():
                c = CPP
                gather_wait(0, c, g0)
                scale(0, c)
                scatter_start(0, c, s0)
                scatter_wait(0, c, s0)

            @pl.when(jnp.logical_not(has_tail))
            def _():
                scatter_wait(0, CPP - 2, s0)

            scatter_wait(1, CPP - 1, s1)

        plsc.subcore_barrier()
        pltpu.sync_copy(acc.at[pl.ds(r0, RPS)], out_hbm.at[cid, pl.ds(r0, RPS)])

        @pl.when(sid == 0)
        def _():
            pltpu.sync_copy(acc.at[pl.ds(RPS * NS, TAIL)],
                            out_hbm.at[cid, pl.ds(RPS * NS, TAIL)])

    return sc_scatter


_sc_scatter_h = _make_sc_scatter(H)
_sc_scatter_c = _make_sc_scatter(C)


def kernel(x, edge_index, edge_weight, W1, b1, W2, b2):
    src = edge_index[0].astype(jnp.int32)
    dst = edge_index[1].astype(jnp.int32)
    ew = edge_weight.astype(jnp.float32)
    kh, kc = KBY[H], KBY[C]
    zh = jnp.zeros((N, H), jnp.float32)
    zc = jnp.zeros((N, C), jnp.float32)

    h1 = _matmul(x, W1)
    p1 = _sc_scatter_h(h1, src.reshape(-1, kh), dst.reshape(-1, kh),
                       ew.reshape(-1, kh), zh)
    h2 = _layer2(p1, b1, W2)
    p2 = _sc_scatter_c(h2, src.reshape(-1, kc), dst.reshape(-1, kc),
                       ew.reshape(-1, kc), zc)
    return _softmax(p2, b2)
